# baseline (device time: 300668 ns/iter reference)
import jax
import jax.numpy as jnp
from jax import lax
from jax.experimental import pallas as pl
from jax.experimental.pallas import tpu as pltpu

K = 32
J = K // 2


def kernel(x):
    m, n = x.shape
    assert m % K == 0
    c = m // K
    xb = x.astype(jnp.bfloat16)

    def body(xb_ref, out_ref, recv_ref,
             xs_sems, xr_sems, fs_sems, yr_sems,
             xv, rv, ov, cpx_sems, cpr_sems, cpo_sems):
        mx = lax.axis_index("x")
        my = lax.axis_index("y")
        mz = lax.axis_index("z")
        xpeer = (1 - mx, my, mz)
        ypeer = (mx, 1 - my, mz)

        barrier = pltpu.get_barrier_semaphore()
        for nbr in (xpeer, ypeer):
            pl.semaphore_signal(barrier, inc=1, device_id=nbr,
                                device_id_type=pl.DeviceIdType.MESH)
        pl.semaphore_wait(barrier, 2)

        def chunk(ref, t):
            return ref.at[pl.ds(t * c, c), :]

        def direct_id(j):
            return 2 * j + my

        def fwd_id(j):
            return 2 * j + (1 - my)

        xsends = []
        for j in range(J):
            t = direct_id(j)
            rdma = pltpu.make_async_remote_copy(
                src_ref=chunk(xb_ref, t),
                dst_ref=chunk(recv_ref, t),
                send_sem=xs_sems.at[j],
                recv_sem=xr_sems.at[j],
                device_id=xpeer,
                device_id_type=pl.DeviceIdType.MESH,
            )
            rdma.start()
            xsends.append(rdma)

        yrecvs = []
        for j in range(J):
            t = fwd_id(j)
            yrecvs.append(pltpu.make_async_remote_copy(
                src_ref=chunk(recv_ref, t),
                dst_ref=chunk(recv_ref, t),
                send_sem=fs_sems.at[j],
                recv_sem=yr_sems.at[j],
                device_id=ypeer,
                device_id_type=pl.DeviceIdType.MESH,
            ))

        items = []
        for j in range(J):
            items.append(("d", j))
            items.append(("f", j))

        fwds = []
        pending = None
        store_chunk = [None, None]
        for idx, (kind, j) in enumerate(items):
            s = idx % 2
            if kind == "d":
                xsends[j].wait_recv()
                t = direct_id(j)
                fwd = pltpu.make_async_remote_copy(
                    src_ref=chunk(recv_ref, t),
                    dst_ref=chunk(recv_ref, t),
                    send_sem=fs_sems.at[j],
                    recv_sem=yr_sems.at[j],
                    device_id=ypeer,
                    device_id_type=pl.DeviceIdType.MESH,
                )
                fwd.start()
                fwds.append(fwd)
            else:
                yrecvs[j].wait_recv()
                t = fwd_id(j)
            cx = pltpu.make_async_copy(chunk(xb_ref, t), xv.at[s],
                                       cpx_sems.at[s])
            cr = pltpu.make_async_copy(chunk(recv_ref, t), rv.at[s],
                                       cpr_sems.at[s])
            cx.start()
            cr.start()
            if pending is not None:
                ps, pt = pending
                pltpu.make_async_copy(chunk(xb_ref, pt), xv.at[ps],
                                      cpx_sems.at[ps]).wait()
                pltpu.make_async_copy(chunk(recv_ref, pt), rv.at[ps],
                                      cpr_sems.at[ps]).wait()
                if store_chunk[ps] is not None:
                    pltpu.make_async_copy(
                        ov.at[ps], chunk(out_ref, store_chunk[ps]),
                        cpo_sems.at[ps]).wait()
                ov[ps] = xv[ps] + rv[ps]
                co = pltpu.make_async_copy(ov.at[ps], chunk(out_ref, pt),
                                           cpo_sems.at[ps])
                co.start()
                store_chunk[ps] = pt
            pending = (s, t)
        ps, pt = pending
        pltpu.make_async_copy(chunk(xb_ref, pt), xv.at[ps],
                              cpx_sems.at[ps]).wait()
        pltpu.make_async_copy(chunk(recv_ref, pt), rv.at[ps],
                              cpr_sems.at[ps]).wait()
        if store_chunk[ps] is not None:
            pltpu.make_async_copy(ov.at[ps], chunk(out_ref, store_chunk[ps]),
                                  cpo_sems.at[ps]).wait()
        ov[ps] = xv[ps] + rv[ps]
        co = pltpu.make_async_copy(ov.at[ps], chunk(out_ref, pt),
                                   cpo_sems.at[ps])
        co.start()
        co.wait()
        if store_chunk[1 - ps] is not None:
            pltpu.make_async_copy(ov.at[1 - ps],
                                  chunk(out_ref, store_chunk[1 - ps]),
                                  cpo_sems.at[1 - ps]).wait()

        for j in range(J):
            xsends[j].wait_send()
            fwds[j].wait_send()

    out, _recv = pl.pallas_call(
        body,
        out_shape=(
            jax.ShapeDtypeStruct((m, n), jnp.bfloat16),
            jax.ShapeDtypeStruct((m, n), jnp.bfloat16),
        ),
        in_specs=[pl.BlockSpec(memory_space=pl.ANY)],
        out_specs=(
            pl.BlockSpec(memory_space=pl.ANY),
            pl.BlockSpec(memory_space=pl.ANY),
        ),
        scratch_shapes=[
            pltpu.SemaphoreType.DMA((J,)),
            pltpu.SemaphoreType.DMA((J,)),
            pltpu.SemaphoreType.DMA((J,)),
            pltpu.SemaphoreType.DMA((J,)),
            pltpu.MemorySpace.VMEM((2, c, n), jnp.bfloat16),
            pltpu.MemorySpace.VMEM((2, c, n), jnp.bfloat16),
            pltpu.MemorySpace.VMEM((2, c, n), jnp.bfloat16),
            pltpu.SemaphoreType.DMA((2,)),
            pltpu.SemaphoreType.DMA((2,)),
            pltpu.SemaphoreType.DMA((2,)),
        ],
        compiler_params=pltpu.CompilerParams(collective_id=0),
    )(xb)
    return out


# device time: 228282 ns/iter; 1.3171x vs baseline; 1.3171x over previous
import jax
import jax.numpy as jnp
from jax import lax
from jax.experimental import pallas as pl
from jax.experimental.pallas import tpu as pltpu

K = 32
J = K // 2
P = 4


def kernel(x):
    m, n = x.shape
    assert m % K == 0
    c = m // K

    def body(x_ref, out_ref, recv_ref, xbb_ref,
             xs_sems, xr_sems, fs_sems, yr_sems,
             fv, bv, av, rv, ov, cfl_sems, cst_sems, cp_sems):
        mx = lax.axis_index("x")
        my = lax.axis_index("y")
        mz = lax.axis_index("z")
        xpeer = (1 - mx, my, mz)
        ypeer = (mx, 1 - my, mz)

        barrier = pltpu.get_barrier_semaphore()
        for nbr in (xpeer, ypeer):
            pl.semaphore_signal(barrier, inc=1, device_id=nbr,
                                device_id_type=pl.DeviceIdType.MESH)

        def chunk(ref, t):
            return ref.at[pl.ds(t * c, c), :]

        def direct_id(j):
            return 2 * j + my

        def fwd_id(j):
            return 2 * j + (1 - my)

        xsends = []
        loads = [None] * J
        stores = [None] * J
        for it in range(J + P):
            if it < J:
                t = direct_id(it)
                ld = pltpu.make_async_copy(chunk(x_ref, t), fv.at[it % P],
                                           cfl_sems.at[it % P])
                ld.start()
                loads[it] = ld
            jc = it - (P - 1)
            if 0 <= jc < J:
                loads[jc].wait()
                sb = jc % 2
                bv[sb] = fv[jc % P].astype(jnp.bfloat16)
                st = pltpu.make_async_copy(bv.at[sb],
                                           chunk(xbb_ref, direct_id(jc)),
                                           cst_sems.at[sb])
                st.start()
                stores[jc] = st
            js = it - P
            if 0 <= js < J:
                if js == 0:
                    pl.semaphore_wait(barrier, 2)
                stores[js].wait()
                t = direct_id(js)
                rdma = pltpu.make_async_remote_copy(
                    src_ref=chunk(xbb_ref, t),
                    dst_ref=chunk(recv_ref, t),
                    send_sem=xs_sems.at[js],
                    recv_sem=xr_sems.at[js],
                    device_id=xpeer,
                    device_id_type=pl.DeviceIdType.MESH,
                )
                rdma.start()
                xsends.append(rdma)

        yrecvs = []
        for j in range(J):
            t = fwd_id(j)
            yrecvs.append(pltpu.make_async_remote_copy(
                src_ref=chunk(recv_ref, t),
                dst_ref=chunk(recv_ref, t),
                send_sem=fs_sems.at[j],
                recv_sem=yr_sems.at[j],
                device_id=ypeer,
                device_id_type=pl.DeviceIdType.MESH,
            ))

        def add_chunk(t):
            ca = pltpu.make_async_copy(chunk(x_ref, t), av, cp_sems.at[0])
            cr = pltpu.make_async_copy(chunk(recv_ref, t), rv, cp_sems.at[1])
            ca.start()
            cr.start()
            ca.wait()
            cr.wait()
            ov[...] = av[...].astype(jnp.bfloat16) + rv[...]
            co = pltpu.make_async_copy(ov, chunk(out_ref, t), cp_sems.at[2])
            co.start()
            co.wait()

        fwds = []
        for j in range(J):
            xsends[j].wait_recv()
            t = direct_id(j)
            fwd = pltpu.make_async_remote_copy(
                src_ref=chunk(recv_ref, t),
                dst_ref=chunk(recv_ref, t),
                send_sem=fs_sems.at[j],
                recv_sem=yr_sems.at[j],
                device_id=ypeer,
                device_id_type=pl.DeviceIdType.MESH,
            )
            fwd.start()
            fwds.append(fwd)
            add_chunk(t)
            if j > 0:
                yrecvs[j - 1].wait_recv()
                add_chunk(fwd_id(j - 1))
        yrecvs[J - 1].wait_recv()
        add_chunk(fwd_id(J - 1))

        for j in range(J):
            xsends[j].wait_send()
            fwds[j].wait_send()

    out, _recv, _xbb = pl.pallas_call(
        body,
        out_shape=(
            jax.ShapeDtypeStruct((m, n), jnp.bfloat16),
            jax.ShapeDtypeStruct((m, n), jnp.bfloat16),
            jax.ShapeDtypeStruct((m, n), jnp.bfloat16),
        ),
        in_specs=[pl.BlockSpec(memory_space=pl.ANY)],
        out_specs=(
            pl.BlockSpec(memory_space=pl.ANY),
            pl.BlockSpec(memory_space=pl.ANY),
            pl.BlockSpec(memory_space=pl.ANY),
        ),
        scratch_shapes=[
            pltpu.SemaphoreType.DMA((J,)),
            pltpu.SemaphoreType.DMA((J,)),
            pltpu.SemaphoreType.DMA((J,)),
            pltpu.SemaphoreType.DMA((J,)),
            pltpu.MemorySpace.VMEM((P, c, n), jnp.float32),
            pltpu.MemorySpace.VMEM((2, c, n), jnp.bfloat16),
            pltpu.MemorySpace.VMEM((c, n), jnp.float32),
            pltpu.MemorySpace.VMEM((c, n), jnp.bfloat16),
            pltpu.MemorySpace.VMEM((c, n), jnp.bfloat16),
            pltpu.SemaphoreType.DMA((P,)),
            pltpu.SemaphoreType.DMA((2,)),
            pltpu.SemaphoreType.DMA((3,)),
        ],
        compiler_params=pltpu.CompilerParams(collective_id=0),
    )(x)
    return out


# device time: 197292 ns/iter; 1.5240x vs baseline; 1.1571x over previous
import jax
import jax.numpy as jnp
from jax import lax
from jax.experimental import pallas as pl
from jax.experimental.pallas import tpu as pltpu

K = 16
ND = 6
P = 3


def kernel(x):
    m, n = x.shape
    assert m % K == 0
    c = m // K

    def body(x_ref, out_ref, recv_ref, xbb_ref,
             xs_sems, xr_sems, zs_sems, zr_sems, ys_sems, yr_sems,
             fv, bv, av, rv, ov, cfl_sems, cst_sems, cp_sems):
        mx = lax.axis_index("x")
        my = lax.axis_index("y")
        mz = lax.axis_index("z")
        a = my
        b = mz % 2
        xpeer = (1 - mx, my, mz)
        ypeer = (mx, 1 - my, mz)
        zpartner = (mx, my, mz + 1 - 2 * b)

        barrier = pltpu.get_barrier_semaphore()
        for nbr in (xpeer, ypeer, zpartner):
            pl.semaphore_signal(barrier, inc=1, device_id=nbr,
                                device_id_type=pl.DeviceIdType.MESH)

        def chunk(ref, t):
            return ref.at[pl.ds(t * c, c), :]

        def direct_list(aa, bb):
            u0 = 2 + bb + 4 * aa
            return [u0, bb, 4 + bb, u0 + 8, 8 + bb, 12 + bb]

        dlist = direct_list(a, b)
        zlist = direct_list(a, 1 - b)
        yU = [2 + b + 4 * (1 - a), 10 + b + 4 * (1 - a)]
        yR = [2 + (1 - b) + 4 * (1 - a), 10 + (1 - b) + 4 * (1 - a)]

        xsends = []
        loads = [None] * ND
        stores = [None] * ND
        for it in range(ND + P):
            if it < ND:
                ld = pltpu.make_async_copy(chunk(x_ref, dlist[it]),
                                           fv.at[it % P],
                                           cfl_sems.at[it % P])
                ld.start()
                loads[it] = ld
            jc = it - (P - 1)
            if 0 <= jc < ND:
                loads[jc].wait()
                sb = jc % 2
                bv[sb] = fv[jc % P].astype(jnp.bfloat16)
                st = pltpu.make_async_copy(bv.at[sb],
                                           chunk(xbb_ref, dlist[jc]),
                                           cst_sems.at[sb])
                st.start()
                stores[jc] = st
            js = it - P
            if 0 <= js < ND:
                if js == 0:
                    pl.semaphore_wait(barrier, 3)
                stores[js].wait()
                rdma = pltpu.make_async_remote_copy(
                    src_ref=chunk(xbb_ref, dlist[js]),
                    dst_ref=chunk(recv_ref, dlist[js]),
                    send_sem=xs_sems.at[js],
                    recv_sem=xr_sems.at[js],
                    device_id=xpeer,
                    device_id_type=pl.DeviceIdType.MESH,
                )
                rdma.start()
                xsends.append(rdma)

        def recv_only(t, sems, slot, peer):
            return pltpu.make_async_remote_copy(
                src_ref=chunk(recv_ref, t),
                dst_ref=chunk(recv_ref, t),
                send_sem=sems.at[slot],
                recv_sem=sems.at[slot],
                device_id=peer,
                device_id_type=pl.DeviceIdType.MESH,
            )

        def forward(t, send_sems, recv_sems, slot, peer):
            f = pltpu.make_async_remote_copy(
                src_ref=chunk(recv_ref, t),
                dst_ref=chunk(recv_ref, t),
                send_sem=send_sems.at[slot],
                recv_sem=recv_sems.at[slot],
                device_id=peer,
                device_id_type=pl.DeviceIdType.MESH,
            )
            f.start()
            return f

        def add_chunk(t):
            ca = pltpu.make_async_copy(chunk(x_ref, t), av, cp_sems.at[0])
            cr = pltpu.make_async_copy(chunk(recv_ref, t), rv, cp_sems.at[1])
            ca.start()
            cr.start()
            ca.wait()
            cr.wait()
            ov[...] = av[...].astype(jnp.bfloat16) + rv[...]
            co = pltpu.make_async_copy(ov, chunk(out_ref, t), cp_sems.at[2])
            co.start()
            co.wait()

        fwds = []

        def on_x(i):
            xsends[i].wait_recv()
            t = dlist[i]
            fwds.append(forward(t, zs_sems, zr_sems, i, zpartner))
            if i % 3 == 0:
                j = i // 3
                fwds.append(forward(t, ys_sems, yr_sems, j, ypeer))
            add_chunk(t)

        def on_z(i):
            recv_only(zlist[i], zr_sems, i, zpartner).wait_recv()
            t = zlist[i]
            if i % 3 == 0:
                j = i // 3
                fwds.append(forward(t, ys_sems, yr_sems, 2 + j, ypeer))
            add_chunk(t)

        def on_yU(j):
            recv_only(yU[j], yr_sems, j, ypeer).wait_recv()
            add_chunk(yU[j])

        def on_yR(j):
            recv_only(yR[j], yr_sems, 2 + j, ypeer).wait_recv()
            add_chunk(yR[j])

        on_x(0)
        on_x(1); on_z(0); on_yU(0)
        on_x(2); on_z(1); on_yR(0)
        on_x(3); on_z(2)
        on_x(4); on_z(3); on_yU(1)
        on_x(5); on_z(4); on_yR(1)
        on_z(5)

        for r in xsends:
            r.wait_send()
        for f in fwds:
            f.wait_send()

    out, _recv, _xbb = pl.pallas_call(
        body,
        out_shape=(
            jax.ShapeDtypeStruct((m, n), jnp.bfloat16),
            jax.ShapeDtypeStruct((m, n), jnp.bfloat16),
            jax.ShapeDtypeStruct((m, n), jnp.bfloat16),
        ),
        in_specs=[pl.BlockSpec(memory_space=pl.ANY)],
        out_specs=(
            pl.BlockSpec(memory_space=pl.ANY),
            pl.BlockSpec(memory_space=pl.ANY),
            pl.BlockSpec(memory_space=pl.ANY),
        ),
        scratch_shapes=[
            pltpu.SemaphoreType.DMA((ND,)),
            pltpu.SemaphoreType.DMA((ND,)),
            pltpu.SemaphoreType.DMA((ND,)),
            pltpu.SemaphoreType.DMA((ND,)),
            pltpu.SemaphoreType.DMA((4,)),
            pltpu.SemaphoreType.DMA((4,)),
            pltpu.MemorySpace.VMEM((P, c, n), jnp.float32),
            pltpu.MemorySpace.VMEM((2, c, n), jnp.bfloat16),
            pltpu.MemorySpace.VMEM((c, n), jnp.float32),
            pltpu.MemorySpace.VMEM((c, n), jnp.bfloat16),
            pltpu.MemorySpace.VMEM((c, n), jnp.bfloat16),
            pltpu.SemaphoreType.DMA((P,)),
            pltpu.SemaphoreType.DMA((2,)),
            pltpu.SemaphoreType.DMA((3,)),
        ],
        compiler_params=pltpu.CompilerParams(collective_id=0),
    )(x)
    return out


# device time: 182508 ns/iter; 1.6474x vs baseline; 1.0810x over previous
import jax
import jax.numpy as jnp
from jax import lax
from jax.experimental import pallas as pl
from jax.experimental.pallas import tpu as pltpu

K = 32
NU = K // 8
ND = 3 * NU
P = 3


def kernel(x):
    m, n = x.shape
    assert m % K == 0
    c = m // K

    def body(x_ref, out_ref, recv_ref, xbb_ref,
             xs_sems, xr_sems, zs_sems, zr_sems, ys_sems, yr_sems,
             fv, bv, av, rv, ov, cfl_sems, cst_sems, cp_sems):
        mx = lax.axis_index("x")
        my = lax.axis_index("y")
        mz = lax.axis_index("z")
        a = my
        b = mz % 2
        xpeer = (1 - mx, my, mz)
        ypeer = (mx, 1 - my, mz)
        zpartner = (mx, my, mz + 1 - 2 * b)

        barrier = pltpu.get_barrier_semaphore()
        for nbr in (xpeer, ypeer, zpartner):
            pl.semaphore_signal(barrier, inc=1, device_id=nbr,
                                device_id_type=pl.DeviceIdType.MESH)

        def chunk(ref, t):
            return ref.at[pl.ds(t * c, c), :]

        def direct_list(aa, bb):
            lst = []
            for j in range(NU):
                lst += [2 + bb + 4 * aa + 8 * j,
                        4 * (2 * j) + bb,
                        4 * (2 * j + 1) + bb]
            return lst

        dlist = direct_list(a, b)
        zlist = direct_list(a, 1 - b)
        yU = [2 + b + 4 * (1 - a) + 8 * j for j in range(NU)]
        yR = [2 + (1 - b) + 4 * (1 - a) + 8 * j for j in range(NU)]

        xsends = []
        loads = [None] * ND
        stores = [None] * ND
        for it in range(ND + P):
            if it < ND:
                ld = pltpu.make_async_copy(chunk(x_ref, dlist[it]),
                                           fv.at[it % P],
                                           cfl_sems.at[it % P])
                ld.start()
                loads[it] = ld
            jc = it - (P - 1)
            if 0 <= jc < ND:
                loads[jc].wait()
                sb = jc % 2
                bv[sb] = fv[jc % P].astype(jnp.bfloat16)
                st = pltpu.make_async_copy(bv.at[sb],
                                           chunk(xbb_ref, dlist[jc]),
                                           cst_sems.at[sb])
                st.start()
                stores[jc] = st
            js = it - P
            if 0 <= js < ND:
                if js == 0:
                    pl.semaphore_wait(barrier, 3)
                stores[js].wait()
                rdma = pltpu.make_async_remote_copy(
                    src_ref=chunk(xbb_ref, dlist[js]),
                    dst_ref=chunk(recv_ref, dlist[js]),
                    send_sem=xs_sems.at[js],
                    recv_sem=xr_sems.at[js],
                    device_id=xpeer,
                    device_id_type=pl.DeviceIdType.MESH,
                )
                rdma.start()
                xsends.append(rdma)

        def recv_only(t, sems, slot, peer):
            return pltpu.make_async_remote_copy(
                src_ref=chunk(recv_ref, t),
                dst_ref=chunk(recv_ref, t),
                send_sem=sems.at[slot],
                recv_sem=sems.at[slot],
                device_id=peer,
                device_id_type=pl.DeviceIdType.MESH,
            )

        def forward(t, send_sems, recv_sems, slot, peer):
            f = pltpu.make_async_remote_copy(
                src_ref=chunk(recv_ref, t),
                dst_ref=chunk(recv_ref, t),
                send_sem=send_sems.at[slot],
                recv_sem=recv_sems.at[slot],
                device_id=peer,
                device_id_type=pl.DeviceIdType.MESH,
            )
            f.start()
            return f

        def add_chunk(t):
            ca = pltpu.make_async_copy(chunk(x_ref, t), av, cp_sems.at[0])
            cr = pltpu.make_async_copy(chunk(recv_ref, t), rv, cp_sems.at[1])
            ca.start()
            cr.start()
            ca.wait()
            cr.wait()
            ov[...] = av[...].astype(jnp.bfloat16) + rv[...]
            co = pltpu.make_async_copy(ov, chunk(out_ref, t), cp_sems.at[2])
            co.start()
            co.wait()

        fwds = []

        def on_x(i):
            xsends[i].wait_recv()
            t = dlist[i]
            fwds.append(forward(t, zs_sems, zr_sems, i, zpartner))
            if i % 3 == 0:
                j = i // 3
                fwds.append(forward(t, ys_sems, yr_sems, j, ypeer))
            add_chunk(t)

        def on_z(i):
            recv_only(zlist[i], zr_sems, i, zpartner).wait_recv()
            t = zlist[i]
            if i % 3 == 0:
                j = i // 3
                fwds.append(forward(t, ys_sems, yr_sems, NU + j, ypeer))
            add_chunk(t)

        def on_yU(j):
            recv_only(yU[j], yr_sems, j, ypeer).wait_recv()
            add_chunk(yU[j])

        def on_yR(j):
            recv_only(yR[j], yr_sems, NU + j, ypeer).wait_recv()
            add_chunk(yR[j])

        for i in range(ND):
            on_x(i)
            if i >= 1:
                on_z(i - 1)
                if (i - 1) % 3 == 0:
                    on_yU((i - 1) // 3)
            if i >= 2 and (i - 2) % 3 == 0:
                on_yR((i - 2) // 3)
        on_z(ND - 1)

        for r in xsends:
            r.wait_send()
        for f in fwds:
            f.wait_send()

    out, _recv, _xbb = pl.pallas_call(
        body,
        out_shape=(
            jax.ShapeDtypeStruct((m, n), jnp.bfloat16),
            jax.ShapeDtypeStruct((m, n), jnp.bfloat16),
            jax.ShapeDtypeStruct((m, n), jnp.bfloat16),
        ),
        in_specs=[pl.BlockSpec(memory_space=pl.ANY)],
        out_specs=(
            pl.BlockSpec(memory_space=pl.ANY),
            pl.BlockSpec(memory_space=pl.ANY),
            pl.BlockSpec(memory_space=pl.ANY),
        ),
        scratch_shapes=[
            pltpu.SemaphoreType.DMA((ND,)),
            pltpu.SemaphoreType.DMA((ND,)),
            pltpu.SemaphoreType.DMA((ND,)),
            pltpu.SemaphoreType.DMA((ND,)),
            pltpu.SemaphoreType.DMA((2 * NU,)),
            pltpu.SemaphoreType.DMA((2 * NU,)),
            pltpu.MemorySpace.VMEM((P, c, n), jnp.float32),
            pltpu.MemorySpace.VMEM((2, c, n), jnp.bfloat16),
            pltpu.MemorySpace.VMEM((c, n), jnp.float32),
            pltpu.MemorySpace.VMEM((c, n), jnp.bfloat16),
            pltpu.MemorySpace.VMEM((c, n), jnp.bfloat16),
            pltpu.SemaphoreType.DMA((P,)),
            pltpu.SemaphoreType.DMA((2,)),
            pltpu.SemaphoreType.DMA((3,)),
        ],
        compiler_params=pltpu.CompilerParams(collective_id=0),
    )(x)
    return out
